# Initial kernel scaffold; baseline (speedup 1.0000x reference)
import jax
import jax.numpy as jnp
from jax.experimental import pallas as pl


def kernel(inputs, Wr, br, expert_emb, W1, b1, W2, b2, W3, b3):
    raise NotImplementedError("write your pallas kernel here")


# speedup vs baseline: 1.8572x; 1.8572x over previous
import functools

import jax
import jax.numpy as jnp
from jax.experimental import pallas as pl


def _router_body(x_ref, wr_ref, br_ref, emb_ref,
                 q_ref, probs_ref, w_ref, idx_ref, gates_ref):
    x = x_ref[...]
    q = jnp.dot(x, wr_ref[...], preferred_element_type=jnp.float32) + br_ref[...]
    q_ref[...] = q
    emb = emb_ref[...]
    diff = q[:, None, :] - emb[None, :, :]
    scores = -jnp.sum(diff * diff, axis=-1)

    m = jnp.max(scores, axis=-1, keepdims=True)
    ex = jnp.exp(scores - m)
    probs_ref[...] = ex / jnp.sum(ex, axis=-1, keepdims=True)

    ncols = scores.shape[-1]
    col = jax.lax.broadcasted_iota(jnp.int32, scores.shape, 1)
    s1 = jnp.max(scores, axis=-1, keepdims=True)
    a1 = jnp.min(jnp.where(scores == s1, col, ncols), axis=-1, keepdims=True)
    masked = jnp.where(col == a1, -jnp.inf, scores)
    s2 = jnp.max(masked, axis=-1, keepdims=True)
    a2 = jnp.min(jnp.where(masked == s2, col, ncols), axis=-1, keepdims=True)

    e21 = jnp.exp(s2 - s1)
    g1 = 1.0 / (1.0 + e21)
    g2 = e21 / (1.0 + e21)
    w_ref[...] = (jnp.where(col == a1, g1, 0.0) +
                  jnp.where(col == a2, g2, 0.0))
    idx_ref[...] = jnp.concatenate([a1, a2], axis=-1)
    gates_ref[...] = jnp.concatenate([g1, g2], axis=-1)


def _router(x, Wr, br, expert_emb, tb):
    B, D = x.shape
    EMB = Wr.shape[1]
    E = expert_emb.shape[0]
    nt = B // tb
    return pl.pallas_call(
        _router_body,
        grid=(nt,),
        in_specs=[
            pl.BlockSpec((tb, D), lambda t: (t, 0)),
            pl.BlockSpec((D, EMB), lambda t: (0, 0)),
            pl.BlockSpec((EMB,), lambda t: (0,)),
            pl.BlockSpec((E, EMB), lambda t: (0, 0)),
        ],
        out_specs=[
            pl.BlockSpec((tb, EMB), lambda t: (t, 0)),
            pl.BlockSpec((tb, E), lambda t: (t, 0)),
            pl.BlockSpec((tb, E), lambda t: (t, 0)),
            pl.BlockSpec((tb, 2), lambda t: (t, 0)),
            pl.BlockSpec((tb, 2), lambda t: (t, 0)),
        ],
        out_shape=[
            jax.ShapeDtypeStruct((B, EMB), jnp.float32),
            jax.ShapeDtypeStruct((B, E), jnp.float32),
            jax.ShapeDtypeStruct((B, E), jnp.float32),
            jax.ShapeDtypeStruct((B, 2), jnp.int32),
            jax.ShapeDtypeStruct((B, 2), jnp.float32),
        ],
    )(x, Wr, br, expert_emb)


def _experts_body(x_ref, w1_ref, b1_ref, w2_ref, b2_ref, w3_ref, b3_ref,
                  wmix_ref, out_ref):
    e = pl.program_id(1)
    x = x_ref[...]
    h1 = jnp.maximum(
        jnp.dot(x, w1_ref[0], preferred_element_type=jnp.float32) + b1_ref[0], 0.0)
    h2 = jnp.maximum(
        jnp.dot(h1, w2_ref[0], preferred_element_type=jnp.float32) + b2_ref[0], 0.0)
    y = jnp.dot(h2, w3_ref[0], preferred_element_type=jnp.float32) + b3_ref[0]
    wmix = wmix_ref[...]
    col = jax.lax.broadcasted_iota(jnp.int32, wmix.shape, 1)
    wcol = jnp.sum(jnp.where(col == e, wmix, 0.0), axis=-1, keepdims=True)
    contrib = y * wcol

    @pl.when(e == 0)
    def _init():
        out_ref[...] = contrib

    @pl.when(e != 0)
    def _acc():
        out_ref[...] += contrib


def _experts(x, W1, b1, W2, b2, W3, b3, wmix, tb):
    B, D = x.shape
    E, _, H = W1.shape
    Ho = W2.shape[2]
    C = W3.shape[2]
    nt = B // tb
    return pl.pallas_call(
        _experts_body,
        grid=(nt, E),
        in_specs=[
            pl.BlockSpec((tb, D), lambda t, e: (t, 0)),
            pl.BlockSpec((1, D, H), lambda t, e: (e, 0, 0)),
            pl.BlockSpec((1, 1, H), lambda t, e: (e, 0, 0)),
            pl.BlockSpec((1, H, Ho), lambda t, e: (e, 0, 0)),
            pl.BlockSpec((1, 1, Ho), lambda t, e: (e, 0, 0)),
            pl.BlockSpec((1, Ho, C), lambda t, e: (e, 0, 0)),
            pl.BlockSpec((1, 1, C), lambda t, e: (e, 0, 0)),
            pl.BlockSpec((tb, E), lambda t, e: (t, 0)),
        ],
        out_specs=pl.BlockSpec((tb, C), lambda t, e: (t, 0)),
        out_shape=jax.ShapeDtypeStruct((B, C), jnp.float32),
    )(x, W1, b1[:, None, :], W2, b2[:, None, :], W3, b3[:, None, :], wmix)


def kernel(inputs, Wr, br, expert_emb, W1, b1, W2, b2, W3, b3):
    B = inputs.shape[0]
    tb = 512 if B % 512 == 0 else B
    query, probs, wmix, _idx2, _gates2 = _router(inputs, Wr, br, expert_emb, tb)
    combined = _experts(inputs, W1, b1, W2, b2, W3, b3, wmix, tb)
    return combined, query, probs
